# baseline (speedup 1.0000x reference)
import jax
import jax.numpy as jnp
from jax import lax
from jax.experimental import pallas as pl
from jax.experimental.pallas import tpu as pltpu
from jax.experimental.pallas import tpu_sc as plsc

B, H, N, D = 8, 16, 8192, 64
KEEP = 6144
EVICT = 2048


def _sort_body(jt_ref, kt_ref, s_in_ref, ss_ref, si_ref):
    step = pl.program_id(0)

    @pl.when(step == 0)
    def _():
        ss_ref[...] = s_in_ref[...]
        si_ref[...] = lax.broadcasted_iota(jnp.int32, (B, N), 1)

    j = jt_ref[step]
    k = kt_ref[step]
    s = ss_ref[...]
    ix = si_ref[...]
    iota = lax.broadcasted_iota(jnp.int32, (B, N), 1)
    bitj = (iota & j) == 0
    sp = jnp.where(bitj, pltpu.roll(s, N - j, 1), pltpu.roll(s, j, 1))
    ip = jnp.where(bitj, pltpu.roll(ix, N - j, 1), pltpu.roll(ix, j, 1))
    up = (iota & k) == 0
    lt_peer = (sp > s) | ((sp == s) & (ip < ix))
    take = lt_peer == (bitj == up)
    ss_ref[...] = jnp.where(take, sp, s)
    si_ref[...] = jnp.where(take, ip, ix)


def _bitonic_steps():
    js, ks = [], []
    k = 2
    while k <= N:
        j = k // 2
        while j >= 1:
            js.append(j)
            ks.append(k)
            j //= 2
        k *= 2
    return js, ks


def _sort_scores(scores):
    js, ks = _bitonic_steps()
    jt = jnp.asarray(js, dtype=jnp.int32)
    kt = jnp.asarray(ks, dtype=jnp.int32)
    ss, si = pl.pallas_call(
        _sort_body,
        grid=(len(js),),
        in_specs=[
            pl.BlockSpec(memory_space=pltpu.SMEM),
            pl.BlockSpec(memory_space=pltpu.SMEM),
            pl.BlockSpec((B, N), lambda i: (0, 0)),
        ],
        out_specs=[
            pl.BlockSpec((B, N), lambda i: (0, 0)),
            pl.BlockSpec((B, N), lambda i: (0, 0)),
        ],
        out_shape=[
            jax.ShapeDtypeStruct((B, N), jnp.float32),
            jax.ShapeDtypeStruct((B, N), jnp.int32),
        ],
    )(jt, kt, scores)
    return ss, si


NC, NS = 2, 16
NW = NC * NS
TPW = (B * H) // NW
LANES = 16
TOTR = TPW * D
UNROLL = 8


def _sc_body(ktn, vtn, sidx_hbm, sc_hbm,
             kkn, kvn, ekn, evn, ks, es,
             sidx_v, krow0, krow1, vrow0, vrow1,
             kob0, kob1, vob0, vob1, srow_v, sob_v,
             rsem0, rsem1, wsem0, wsem1):
    c = lax.axis_index("c")
    s = lax.axis_index("s")
    wid = s * NC + c
    b = wid // TPW

    krow = (krow0, krow1)
    vrow = (vrow0, vrow1)
    kob = (kob0, kob1)
    vob = (vob0, vob1)
    rsem = (rsem0, rsem1)
    wsem = (wsem0, wsem1)

    pltpu.sync_copy(sidx_hbm.at[b], sidx_v)

    def rowrefs(r):
        t = r // D
        d = r % D
        bh = wid * TPW + t
        b4 = bh // H
        h4 = bh % H
        return b4, h4, d

    def prefetch(r, p):
        b4, h4, d = rowrefs(r)
        pltpu.async_copy(ktn.at[b4, h4, d], krow[p], rsem[p])
        pltpu.async_copy(vtn.at[b4, h4, d], vrow[p], rsem[p])

    prefetch(0, 0)

    def half(r, p):
        b4, h4, d = rowrefs(r)

        @pl.when(r < TOTR - 1)
        def _():
            prefetch(r + 1, 1 - p)

        pltpu.make_async_copy(ktn.at[0, 0, 0], krow[p], rsem[p]).wait()
        pltpu.make_async_copy(ktn.at[0, 0, 0], vrow[p], rsem[p]).wait()

        @pl.when(r >= 2)
        def _():
            pltpu.make_async_copy(kob[p], kkn.at[0, 0, 0], wsem[p]).wait()
            pltpu.make_async_copy(kob[p], ekn.at[0, 0, 0], wsem[p]).wait()
            pltpu.make_async_copy(vob[p], kkn.at[0, 0, 0], wsem[p]).wait()
            pltpu.make_async_copy(vob[p], ekn.at[0, 0, 0], wsem[p]).wait()

        def gat(o, carry):
            sl = pl.ds(o * LANES, LANES)
            iv = sidx_v[sl]
            kob[p][sl] = plsc.load_gather(krow[p], [iv])
            vob[p][sl] = plsc.load_gather(vrow[p], [iv])
            return carry

        lax.fori_loop(0, N // LANES, gat, 0, unroll=UNROLL)

        pltpu.async_copy(kob[p].at[pl.ds(0, KEEP)], kkn.at[b4, h4, d], wsem[p])
        pltpu.async_copy(kob[p].at[pl.ds(KEEP, EVICT)], ekn.at[b4, h4, d], wsem[p])
        pltpu.async_copy(vob[p].at[pl.ds(0, KEEP)], kvn.at[b4, h4, d], wsem[p])
        pltpu.async_copy(vob[p].at[pl.ds(KEEP, EVICT)], evn.at[b4, h4, d], wsem[p])

    def outer(r2, carry):
        half(r2 * 2, 0)
        half(r2 * 2 + 1, 1)
        return carry

    lax.fori_loop(0, TOTR // 2, outer, 0)

    @pl.when(wid % TPW == 0)
    def _():
        pltpu.sync_copy(sc_hbm.at[b], srow_v)

        def sgat(o, carry):
            sl = pl.ds(o * LANES, LANES)
            sob_v[sl] = plsc.load_gather(srow_v, [sidx_v[sl]])
            return carry

        lax.fori_loop(0, N // LANES, sgat, 0, unroll=UNROLL)
        pltpu.sync_copy(sob_v.at[pl.ds(0, KEEP)], ks.at[b])
        pltpu.sync_copy(sob_v.at[pl.ds(KEEP, EVICT)], es.at[b])

    for p in range(2):
        pltpu.make_async_copy(kob[p], kkn.at[0, 0, 0], wsem[p]).wait()
        pltpu.make_async_copy(kob[p], ekn.at[0, 0, 0], wsem[p]).wait()
        pltpu.make_async_copy(vob[p], kkn.at[0, 0, 0], wsem[p]).wait()
        pltpu.make_async_copy(vob[p], ekn.at[0, 0, 0], wsem[p]).wait()


def _make_sc_gather():
    return pl.kernel(
        _sc_body,
        out_type=(
            jax.ShapeDtypeStruct((B, H, D, KEEP), jnp.float32),
            jax.ShapeDtypeStruct((B, H, D, KEEP), jnp.float32),
            jax.ShapeDtypeStruct((B, H, D, EVICT), jnp.float32),
            jax.ShapeDtypeStruct((B, H, D, EVICT), jnp.float32),
            jax.ShapeDtypeStruct((B, KEEP), jnp.float32),
            jax.ShapeDtypeStruct((B, EVICT), jnp.float32),
        ),
        mesh=plsc.VectorSubcoreMesh(
            core_axis_name="c", subcore_axis_name="s",
            num_cores=NC, num_subcores=NS),
        compiler_params=pltpu.CompilerParams(
            use_tc_tiling_on_sc=False, needs_layout_passes=False),
        scratch_types=[
            pltpu.VMEM((N,), jnp.int32),
            pltpu.VMEM((N,), jnp.float32),
            pltpu.VMEM((N,), jnp.float32),
            pltpu.VMEM((N,), jnp.float32),
            pltpu.VMEM((N,), jnp.float32),
            pltpu.VMEM((N,), jnp.float32),
            pltpu.VMEM((N,), jnp.float32),
            pltpu.VMEM((N,), jnp.float32),
            pltpu.VMEM((N,), jnp.float32),
            pltpu.VMEM((N,), jnp.float32),
            pltpu.VMEM((N,), jnp.float32),
            pltpu.SemaphoreType.DMA,
            pltpu.SemaphoreType.DMA,
            pltpu.SemaphoreType.DMA,
            pltpu.SemaphoreType.DMA,
        ],
    )


def kernel(k, v, scores, n_evict):
    del n_evict
    _, sidx = _sort_scores(scores)
    ktn = jnp.swapaxes(k, 2, 3)
    vtn = jnp.swapaxes(v, 2, 3)
    kkn, kvn, ekn, evn, ks, es = _make_sc_gather()(ktn, vtn, sidx, scores)
    return (
        jnp.swapaxes(kkn, 2, 3),
        jnp.swapaxes(kvn, 2, 3),
        ks,
        jnp.swapaxes(ekn, 2, 3),
        jnp.swapaxes(evn, 2, 3),
        es,
    )


# speedup vs baseline: 2.0226x; 2.0226x over previous
import jax
import jax.numpy as jnp
from jax import lax
from jax.experimental import pallas as pl
from jax.experimental.pallas import tpu as pltpu
from jax.experimental.pallas import tpu_sc as plsc

B, H, N, D = 8, 16, 8192, 64
KEEP = 6144
EVICT = 2048


def _sort_body(jt_ref, kt_ref, s_in_ref, ss_ref, si_ref):
    step = pl.program_id(0)

    @pl.when(step == 0)
    def _():
        ss_ref[...] = s_in_ref[...]
        si_ref[...] = lax.broadcasted_iota(jnp.int32, (B, N), 1)

    j = jt_ref[step]
    k = kt_ref[step]
    s = ss_ref[...]
    ix = si_ref[...]
    iota = lax.broadcasted_iota(jnp.int32, (B, N), 1)
    bitj = (iota & j) == 0
    sp = jnp.where(bitj, pltpu.roll(s, N - j, 1), pltpu.roll(s, j, 1))
    ip = jnp.where(bitj, pltpu.roll(ix, N - j, 1), pltpu.roll(ix, j, 1))
    up = (iota & k) == 0
    lt_peer = (sp > s) | ((sp == s) & (ip < ix))
    take = lt_peer == (bitj == up)
    ss_ref[...] = jnp.where(take, sp, s)
    si_ref[...] = jnp.where(take, ip, ix)


def _bitonic_steps():
    js, ks = [], []
    k = 2
    while k <= N:
        j = k // 2
        while j >= 1:
            js.append(j)
            ks.append(k)
            j //= 2
        k *= 2
    return js, ks


def _sort_scores(scores):
    js, ks = _bitonic_steps()
    jt = jnp.asarray(js, dtype=jnp.int32)
    kt = jnp.asarray(ks, dtype=jnp.int32)
    ss, si = pl.pallas_call(
        _sort_body,
        grid=(len(js),),
        in_specs=[
            pl.BlockSpec(memory_space=pltpu.SMEM),
            pl.BlockSpec(memory_space=pltpu.SMEM),
            pl.BlockSpec((B, N), lambda i: (0, 0)),
        ],
        out_specs=[
            pl.BlockSpec((B, N), lambda i: (0, 0)),
            pl.BlockSpec((B, N), lambda i: (0, 0)),
        ],
        out_shape=[
            jax.ShapeDtypeStruct((B, N), jnp.float32),
            jax.ShapeDtypeStruct((B, N), jnp.int32),
        ],
    )(jt, kt, scores)
    return ss, si


NC, NS = 2, 16
NW = NC * NS
TPW = (B * H) // NW
LANES = 16
TOTR = TPW * D
UNROLL = 8


def _sc_body(ktn, vtn, sidx_hbm, sc_hbm,
             kkn, kvn, ekn, evn, ks, es,
             sidx_v, krow0, krow1, vrow0, vrow1,
             kob0, kob1, vob0, vob1, srow_v, sob_v,
             rsem0, rsem1, wsem0, wsem1):
    c = lax.axis_index("c")
    s = lax.axis_index("s")
    wid = s * NC + c
    b = wid // TPW

    krow = (krow0, krow1)
    vrow = (vrow0, vrow1)
    kob = (kob0, kob1)
    vob = (vob0, vob1)
    rsem = (rsem0, rsem1)
    wsem = (wsem0, wsem1)

    pltpu.sync_copy(sidx_hbm.at[b], sidx_v)

    def rowrefs(r):
        t = r // D
        d = r % D
        bh = wid * TPW + t
        b4 = bh // H
        h4 = bh % H
        return b4, h4, d

    def prefetch(r, p):
        b4, h4, d = rowrefs(r)
        pltpu.async_copy(ktn.at[b4, h4, d], krow[p], rsem[p])
        pltpu.async_copy(vtn.at[b4, h4, d], vrow[p], rsem[p])

    prefetch(0, 0)

    def half(r, p):
        b4, h4, d = rowrefs(r)

        @pl.when(r < TOTR - 1)
        def _():
            prefetch(r + 1, 1 - p)

        pltpu.make_async_copy(ktn.at[0, 0, 0], krow[p], rsem[p]).wait()
        pltpu.make_async_copy(ktn.at[0, 0, 0], vrow[p], rsem[p]).wait()

        @pl.when(r >= 2)
        def _():
            pltpu.make_async_copy(kob[p], kkn.at[0, 0, 0], wsem[p]).wait()
            pltpu.make_async_copy(kob[p], ekn.at[0, 0, 0], wsem[p]).wait()
            pltpu.make_async_copy(vob[p], kkn.at[0, 0, 0], wsem[p]).wait()
            pltpu.make_async_copy(vob[p], ekn.at[0, 0, 0], wsem[p]).wait()

        @plsc.parallel_loop(0, N, step=LANES, unroll=UNROLL)
        def _(o):
            sl = pl.ds(o, LANES)
            iv = sidx_v[sl]
            kob[p][sl] = plsc.load_gather(krow[p], [iv])
            vob[p][sl] = plsc.load_gather(vrow[p], [iv])

        pltpu.async_copy(kob[p].at[pl.ds(0, KEEP)], kkn.at[b4, h4, d], wsem[p])
        pltpu.async_copy(kob[p].at[pl.ds(KEEP, EVICT)], ekn.at[b4, h4, d], wsem[p])
        pltpu.async_copy(vob[p].at[pl.ds(0, KEEP)], kvn.at[b4, h4, d], wsem[p])
        pltpu.async_copy(vob[p].at[pl.ds(KEEP, EVICT)], evn.at[b4, h4, d], wsem[p])

    def outer(r2, carry):
        half(r2 * 2, 0)
        half(r2 * 2 + 1, 1)
        return carry

    lax.fori_loop(0, TOTR // 2, outer, 0)

    @pl.when(wid % TPW == 0)
    def _():
        pltpu.sync_copy(sc_hbm.at[b], srow_v)

        @plsc.parallel_loop(0, N, step=LANES, unroll=UNROLL)
        def _(o):
            sl = pl.ds(o, LANES)
            sob_v[sl] = plsc.load_gather(srow_v, [sidx_v[sl]])
        pltpu.sync_copy(sob_v.at[pl.ds(0, KEEP)], ks.at[b])
        pltpu.sync_copy(sob_v.at[pl.ds(KEEP, EVICT)], es.at[b])

    for p in range(2):
        pltpu.make_async_copy(kob[p], kkn.at[0, 0, 0], wsem[p]).wait()
        pltpu.make_async_copy(kob[p], ekn.at[0, 0, 0], wsem[p]).wait()
        pltpu.make_async_copy(vob[p], kkn.at[0, 0, 0], wsem[p]).wait()
        pltpu.make_async_copy(vob[p], ekn.at[0, 0, 0], wsem[p]).wait()


def _make_sc_gather():
    return pl.kernel(
        _sc_body,
        out_type=(
            jax.ShapeDtypeStruct((B, H, D, KEEP), jnp.float32),
            jax.ShapeDtypeStruct((B, H, D, KEEP), jnp.float32),
            jax.ShapeDtypeStruct((B, H, D, EVICT), jnp.float32),
            jax.ShapeDtypeStruct((B, H, D, EVICT), jnp.float32),
            jax.ShapeDtypeStruct((B, KEEP), jnp.float32),
            jax.ShapeDtypeStruct((B, EVICT), jnp.float32),
        ),
        mesh=plsc.VectorSubcoreMesh(
            core_axis_name="c", subcore_axis_name="s",
            num_cores=NC, num_subcores=NS),
        compiler_params=pltpu.CompilerParams(
            use_tc_tiling_on_sc=False, needs_layout_passes=False),
        scratch_types=[
            pltpu.VMEM((N,), jnp.int32),
            pltpu.VMEM((N,), jnp.float32),
            pltpu.VMEM((N,), jnp.float32),
            pltpu.VMEM((N,), jnp.float32),
            pltpu.VMEM((N,), jnp.float32),
            pltpu.VMEM((N,), jnp.float32),
            pltpu.VMEM((N,), jnp.float32),
            pltpu.VMEM((N,), jnp.float32),
            pltpu.VMEM((N,), jnp.float32),
            pltpu.VMEM((N,), jnp.float32),
            pltpu.VMEM((N,), jnp.float32),
            pltpu.SemaphoreType.DMA,
            pltpu.SemaphoreType.DMA,
            pltpu.SemaphoreType.DMA,
            pltpu.SemaphoreType.DMA,
        ],
    )


def kernel(k, v, scores, n_evict):
    del n_evict
    _, sidx = _sort_scores(scores)
    ktn = jnp.swapaxes(k, 2, 3)
    vtn = jnp.swapaxes(v, 2, 3)
    kkn, kvn, ekn, evn, ks, es = _make_sc_gather()(ktn, vtn, sidx, scores)
    return (
        jnp.swapaxes(kkn, 2, 3),
        jnp.swapaxes(kvn, 2, 3),
        ks,
        jnp.swapaxes(ekn, 2, 3),
        jnp.swapaxes(evn, 2, 3),
        es,
    )


# speedup vs baseline: 4.9059x; 2.4255x over previous
import jax
import jax.numpy as jnp
from jax import lax
from jax.experimental import pallas as pl
from jax.experimental.pallas import tpu as pltpu
from jax.experimental.pallas import tpu_sc as plsc

B, H, N, D = 8, 16, 8192, 64
KEEP = 6144
EVICT = 2048
NT = N // 128
KT = KEEP // 128
ET = EVICT // 128
DT = D // 8


def _sort_body(jt_ref, kt_ref, s_in_ref, ss_ref, si_ref):
    step = pl.program_id(0)

    @pl.when(step == 0)
    def _():
        ss_ref[...] = s_in_ref[...]
        si_ref[...] = lax.broadcasted_iota(jnp.int32, (B, N), 1)

    j = jt_ref[step]
    k = kt_ref[step]
    s = ss_ref[...]
    ix = si_ref[...]
    iota = lax.broadcasted_iota(jnp.int32, (B, N), 1)
    bitj = (iota & j) == 0
    sp = jnp.where(bitj, pltpu.roll(s, N - j, 1), pltpu.roll(s, j, 1))
    ip = jnp.where(bitj, pltpu.roll(ix, N - j, 1), pltpu.roll(ix, j, 1))
    up = (iota & k) == 0
    lt_peer = (sp > s) | ((sp == s) & (ip < ix))
    take = lt_peer == (bitj == up)
    ss_ref[...] = jnp.where(take, sp, s)
    si_ref[...] = jnp.where(take, ip, ix)


def _bitonic_steps():
    js, ks = [], []
    k = 2
    while k <= N:
        j = k // 2
        while j >= 1:
            js.append(j)
            ks.append(k)
            j //= 2
        k *= 2
    return js, ks


def _sort_scores(scores):
    js, ks = _bitonic_steps()
    jt = jnp.asarray(js, dtype=jnp.int32)
    kt = jnp.asarray(ks, dtype=jnp.int32)
    ss, si = pl.pallas_call(
        _sort_body,
        grid=(len(js),),
        in_specs=[
            pl.BlockSpec(memory_space=pltpu.SMEM),
            pl.BlockSpec(memory_space=pltpu.SMEM),
            pl.BlockSpec((B, N), lambda i: (0, 0)),
        ],
        out_specs=[
            pl.BlockSpec((B, N), lambda i: (0, 0)),
            pl.BlockSpec((B, N), lambda i: (0, 0)),
        ],
        out_shape=[
            jax.ShapeDtypeStruct((B, N), jnp.float32),
            jax.ShapeDtypeStruct((B, N), jnp.int32),
        ],
    )(jt, kt, scores)
    return ss, si


NC, NS = 2, 16
NW = NC * NS
TPW = (B * H) // NW
LANES = 16
TOTR = TPW * D
UNROLL = 8


def _sc_body(k6, v6, sidx_hbm, sc_hbm,
             kk6, kv6, ek6, ev6, ks, es,
             sidx_v, shi_v, slo_v, krow0, krow1, vrow0, vrow1,
             kob0, kob1, vob0, vob1, srow_v, sob_v,
             rsem0, rsem1, wsem0, wsem1):
    c = lax.axis_index("c")
    s = lax.axis_index("s")
    wid = s * NC + c
    b = wid // TPW

    krow = (krow0, krow1)
    vrow = (vrow0, vrow1)
    kob = (kob0, kob1)
    vob = (vob0, vob1)
    rsem = (rsem0, rsem1)
    wsem = (wsem0, wsem1)

    pltpu.sync_copy(sidx_hbm.at[b], sidx_v)

    @plsc.parallel_loop(0, N, step=LANES)
    def _(o):
        sl = pl.ds(o, LANES)
        iv = sidx_v[sl]
        shi_v[sl] = iv >> 7
        slo_v[sl] = iv & 127

    def rowrefs(r):
        t = r // D
        d = r % D
        bh = wid * TPW + t
        return bh // H, bh % H, d // 8, d % 8

    def prefetch(r, p):
        b4, h4, dt, dr = rowrefs(r)
        pltpu.async_copy(k6.at[b4, h4, dt, :, dr], krow[p], rsem[p])
        pltpu.async_copy(v6.at[b4, h4, dt, :, dr], vrow[p], rsem[p])

    prefetch(0, 0)

    def half(r, p):
        b4, h4, dt, dr = rowrefs(r)

        @pl.when(r < TOTR - 1)
        def _():
            prefetch(r + 1, 1 - p)

        pltpu.make_async_copy(k6.at[0, 0, 0, :, 0], krow[p], rsem[p]).wait()
        pltpu.make_async_copy(k6.at[0, 0, 0, :, 0], vrow[p], rsem[p]).wait()

        @pl.when(r >= 2)
        def _():
            pltpu.make_async_copy(kob[p], kk6.at[0, 0, 0, :, 0], wsem[p]).wait()
            pltpu.make_async_copy(kob[p], ek6.at[0, 0, 0, :, 0], wsem[p]).wait()
            pltpu.make_async_copy(vob[p], kk6.at[0, 0, 0, :, 0], wsem[p]).wait()
            pltpu.make_async_copy(vob[p], ek6.at[0, 0, 0, :, 0], wsem[p]).wait()

        @plsc.parallel_loop(0, N, step=LANES, unroll=UNROLL)
        def _(o):
            sl = pl.ds(o, LANES)
            hi = shi_v[sl]
            lo = slo_v[sl]
            r2 = o // 128
            c2 = o % 128
            cs = pl.ds(c2, LANES)
            kob[p][r2, cs] = plsc.load_gather(krow[p], [hi, lo])
            vob[p][r2, cs] = plsc.load_gather(vrow[p], [hi, lo])

        pltpu.async_copy(kob[p].at[pl.ds(0, KT)], kk6.at[b4, h4, dt, :, dr],
                         wsem[p])
        pltpu.async_copy(kob[p].at[pl.ds(KT, ET)], ek6.at[b4, h4, dt, :, dr],
                         wsem[p])
        pltpu.async_copy(vob[p].at[pl.ds(0, KT)], kv6.at[b4, h4, dt, :, dr],
                         wsem[p])
        pltpu.async_copy(vob[p].at[pl.ds(KT, ET)], ev6.at[b4, h4, dt, :, dr],
                         wsem[p])

    def outer(r2, carry):
        half(r2 * 2, 0)
        half(r2 * 2 + 1, 1)
        return carry

    lax.fori_loop(0, TOTR // 2, outer, 0)

    @pl.when(wid % TPW == 0)
    def _():
        pltpu.sync_copy(sc_hbm.at[b], srow_v)

        @plsc.parallel_loop(0, N, step=LANES, unroll=UNROLL)
        def _(o):
            sl = pl.ds(o, LANES)
            sob_v[sl] = plsc.load_gather(srow_v, [sidx_v[sl]])

        pltpu.sync_copy(sob_v.at[pl.ds(0, KEEP)], ks.at[b])
        pltpu.sync_copy(sob_v.at[pl.ds(KEEP, EVICT)], es.at[b])

    for p in range(2):
        pltpu.make_async_copy(kob[p], kk6.at[0, 0, 0, :, 0], wsem[p]).wait()
        pltpu.make_async_copy(kob[p], ek6.at[0, 0, 0, :, 0], wsem[p]).wait()
        pltpu.make_async_copy(vob[p], kk6.at[0, 0, 0, :, 0], wsem[p]).wait()
        pltpu.make_async_copy(vob[p], ek6.at[0, 0, 0, :, 0], wsem[p]).wait()


def _make_sc_gather():
    return pl.kernel(
        _sc_body,
        out_type=(
            jax.ShapeDtypeStruct((B, H, DT, KT, 8, 128), jnp.float32),
            jax.ShapeDtypeStruct((B, H, DT, KT, 8, 128), jnp.float32),
            jax.ShapeDtypeStruct((B, H, DT, ET, 8, 128), jnp.float32),
            jax.ShapeDtypeStruct((B, H, DT, ET, 8, 128), jnp.float32),
            jax.ShapeDtypeStruct((B, KEEP), jnp.float32),
            jax.ShapeDtypeStruct((B, EVICT), jnp.float32),
        ),
        mesh=plsc.VectorSubcoreMesh(
            core_axis_name="c", subcore_axis_name="s",
            num_cores=NC, num_subcores=NS),
        compiler_params=pltpu.CompilerParams(
            use_tc_tiling_on_sc=False, needs_layout_passes=False),
        scratch_types=[
            pltpu.VMEM((N,), jnp.int32),
            pltpu.VMEM((N,), jnp.int32),
            pltpu.VMEM((N,), jnp.int32),
            pltpu.VMEM((NT, 128), jnp.float32),
            pltpu.VMEM((NT, 128), jnp.float32),
            pltpu.VMEM((NT, 128), jnp.float32),
            pltpu.VMEM((NT, 128), jnp.float32),
            pltpu.VMEM((NT, 128), jnp.float32),
            pltpu.VMEM((NT, 128), jnp.float32),
            pltpu.VMEM((NT, 128), jnp.float32),
            pltpu.VMEM((NT, 128), jnp.float32),
            pltpu.VMEM((N,), jnp.float32),
            pltpu.VMEM((N,), jnp.float32),
            pltpu.SemaphoreType.DMA,
            pltpu.SemaphoreType.DMA,
            pltpu.SemaphoreType.DMA,
            pltpu.SemaphoreType.DMA,
        ],
    )


def kernel(k, v, scores, n_evict):
    del n_evict
    _, sidx = _sort_scores(scores)
    k6 = jnp.transpose(k.reshape(B, H, NT, 128, DT, 8), (0, 1, 4, 2, 5, 3))
    v6 = jnp.transpose(v.reshape(B, H, NT, 128, DT, 8), (0, 1, 4, 2, 5, 3))
    kk6, kv6, ek6, ev6, ks, es = _make_sc_gather()(k6, v6, sidx, scores)

    def back(x6, S):
        return jnp.transpose(x6, (0, 1, 3, 5, 2, 4)).reshape(B, H, S, D)

    return (back(kk6, KEEP), back(kv6, KEEP), ks,
            back(ek6, EVICT), back(ev6, EVICT), es)


# speedup vs baseline: 5.2181x; 1.0637x over previous
import jax
import jax.numpy as jnp
from jax import lax
from jax.experimental import pallas as pl
from jax.experimental.pallas import tpu as pltpu
from jax.experimental.pallas import tpu_sc as plsc

B, H, N, D = 8, 16, 8192, 64
KEEP = 6144
EVICT = 2048
NT = N // 128
KT = KEEP // 128
ET = EVICT // 128
DT = D // 8


def _sort_body(jt_ref, kt_ref, s_in_ref, ss_ref, si_ref):
    step = pl.program_id(0)

    @pl.when(step == 0)
    def _():
        ss_ref[...] = s_in_ref[...]
        si_ref[...] = lax.broadcasted_iota(jnp.int32, (B, N), 1)

    j = jt_ref[step]
    k = kt_ref[step]
    s = ss_ref[...]
    ix = si_ref[...]
    iota = lax.broadcasted_iota(jnp.int32, (B, N), 1)
    bitj = (iota & j) == 0
    sp = jnp.where(bitj, pltpu.roll(s, N - j, 1), pltpu.roll(s, j, 1))
    ip = jnp.where(bitj, pltpu.roll(ix, N - j, 1), pltpu.roll(ix, j, 1))
    up = (iota & k) == 0
    lt_peer = (sp > s) | ((sp == s) & (ip < ix))
    take = lt_peer == (bitj == up)
    ss_ref[...] = jnp.where(take, sp, s)
    si_ref[...] = jnp.where(take, ip, ix)


def _bitonic_steps():
    js, ks = [], []
    k = 2
    while k <= N:
        j = k // 2
        while j >= 1:
            js.append(j)
            ks.append(k)
            j //= 2
        k *= 2
    return js, ks


def _sort_scores(scores):
    js, ks = _bitonic_steps()
    jt = jnp.asarray(js, dtype=jnp.int32)
    kt = jnp.asarray(ks, dtype=jnp.int32)
    ss, si = pl.pallas_call(
        _sort_body,
        grid=(len(js),),
        in_specs=[
            pl.BlockSpec(memory_space=pltpu.SMEM),
            pl.BlockSpec(memory_space=pltpu.SMEM),
            pl.BlockSpec((B, N), lambda i: (0, 0)),
        ],
        out_specs=[
            pl.BlockSpec((B, N), lambda i: (0, 0)),
            pl.BlockSpec((B, N), lambda i: (0, 0)),
        ],
        out_shape=[
            jax.ShapeDtypeStruct((B, N), jnp.float32),
            jax.ShapeDtypeStruct((B, N), jnp.int32),
        ],
    )(jt, kt, scores)
    return ss, si


NC, NS = 2, 16
NW = NC * NS
TPW = (B * H) // NW
LANES = 16
TOTR = TPW * D
UNROLL = 8


def _sc_body(k6, v6, sidx_hbm, sc_hbm,
             kk6, kv6, ek6, ev6, ks, es,
             sidx_v, krow0, krow1, vrow0, vrow1,
             kob0, kob1, vob0, vob1, srow_v, sob_v,
             rsem0, rsem1, wsem0, wsem1):
    c = lax.axis_index("c")
    s = lax.axis_index("s")
    wid = s * NC + c
    b = wid // TPW

    krow = (krow0, krow1)
    vrow = (vrow0, vrow1)
    kob = (kob0, kob1)
    vob = (vob0, vob1)
    rsem = (rsem0, rsem1)
    wsem = (wsem0, wsem1)

    pltpu.sync_copy(sidx_hbm.at[b], sidx_v)

    def rowrefs(r):
        t = r // D
        d = r % D
        bh = wid * TPW + t
        return bh // H, bh % H, d // 8, d % 8

    def prefetch(r, p):
        b4, h4, dt, dr = rowrefs(r)
        pltpu.async_copy(k6.at[b4, h4, dt, :, dr], krow[p], rsem[p])
        pltpu.async_copy(v6.at[b4, h4, dt, :, dr], vrow[p], rsem[p])

    prefetch(0, 0)

    def half(r, p):
        b4, h4, dt, dr = rowrefs(r)

        @pl.when(r < TOTR - 1)
        def _():
            prefetch(r + 1, 1 - p)

        pltpu.make_async_copy(k6.at[0, 0, 0, :, 0], krow[p], rsem[p]).wait()
        pltpu.make_async_copy(k6.at[0, 0, 0, :, 0], vrow[p], rsem[p]).wait()

        @pl.when(r >= 2)
        def _():
            pltpu.make_async_copy(kob[p], kk6.at[0, 0, 0, :, 0], wsem[p]).wait()
            pltpu.make_async_copy(kob[p], ek6.at[0, 0, 0, :, 0], wsem[p]).wait()
            pltpu.make_async_copy(vob[p], kk6.at[0, 0, 0, :, 0], wsem[p]).wait()
            pltpu.make_async_copy(vob[p], ek6.at[0, 0, 0, :, 0], wsem[p]).wait()

        @plsc.parallel_loop(0, N, step=LANES, unroll=UNROLL)
        def _(o):
            sl = pl.ds(o, LANES)
            iv = sidx_v[sl]
            hi = iv >> 7
            lo = iv & 127
            r2 = o // 128
            c2 = o % 128
            cs = pl.ds(c2, LANES)
            kob[p][r2, cs] = plsc.load_gather(krow[p], [hi, lo])
            vob[p][r2, cs] = plsc.load_gather(vrow[p], [hi, lo])

        pltpu.async_copy(kob[p].at[pl.ds(0, KT)], kk6.at[b4, h4, dt, :, dr],
                         wsem[p])
        pltpu.async_copy(kob[p].at[pl.ds(KT, ET)], ek6.at[b4, h4, dt, :, dr],
                         wsem[p])
        pltpu.async_copy(vob[p].at[pl.ds(0, KT)], kv6.at[b4, h4, dt, :, dr],
                         wsem[p])
        pltpu.async_copy(vob[p].at[pl.ds(KT, ET)], ev6.at[b4, h4, dt, :, dr],
                         wsem[p])

    def outer(r2, carry):
        half(r2 * 2, 0)
        half(r2 * 2 + 1, 1)
        return carry

    lax.fori_loop(0, TOTR // 2, outer, 0)

    @pl.when(wid % TPW == 0)
    def _():
        pltpu.sync_copy(sc_hbm.at[b], srow_v)

        @plsc.parallel_loop(0, N, step=LANES, unroll=UNROLL)
        def _(o):
            sl = pl.ds(o, LANES)
            sob_v[sl] = plsc.load_gather(srow_v, [sidx_v[sl]])

        pltpu.sync_copy(sob_v.at[pl.ds(0, KEEP)], ks.at[b])
        pltpu.sync_copy(sob_v.at[pl.ds(KEEP, EVICT)], es.at[b])

    for p in range(2):
        pltpu.make_async_copy(kob[p], kk6.at[0, 0, 0, :, 0], wsem[p]).wait()
        pltpu.make_async_copy(kob[p], ek6.at[0, 0, 0, :, 0], wsem[p]).wait()
        pltpu.make_async_copy(vob[p], kk6.at[0, 0, 0, :, 0], wsem[p]).wait()
        pltpu.make_async_copy(vob[p], ek6.at[0, 0, 0, :, 0], wsem[p]).wait()


def _make_sc_gather():
    return pl.kernel(
        _sc_body,
        out_type=(
            jax.ShapeDtypeStruct((B, H, DT, KT, 8, 128), jnp.float32),
            jax.ShapeDtypeStruct((B, H, DT, KT, 8, 128), jnp.float32),
            jax.ShapeDtypeStruct((B, H, DT, ET, 8, 128), jnp.float32),
            jax.ShapeDtypeStruct((B, H, DT, ET, 8, 128), jnp.float32),
            jax.ShapeDtypeStruct((B, KEEP), jnp.float32),
            jax.ShapeDtypeStruct((B, EVICT), jnp.float32),
        ),
        mesh=plsc.VectorSubcoreMesh(
            core_axis_name="c", subcore_axis_name="s",
            num_cores=NC, num_subcores=NS),
        compiler_params=pltpu.CompilerParams(
            use_tc_tiling_on_sc=False, needs_layout_passes=False),
        scratch_types=[
            pltpu.VMEM((N,), jnp.int32),
            pltpu.VMEM((NT, 128), jnp.float32),
            pltpu.VMEM((NT, 128), jnp.float32),
            pltpu.VMEM((NT, 128), jnp.float32),
            pltpu.VMEM((NT, 128), jnp.float32),
            pltpu.VMEM((NT, 128), jnp.float32),
            pltpu.VMEM((NT, 128), jnp.float32),
            pltpu.VMEM((NT, 128), jnp.float32),
            pltpu.VMEM((NT, 128), jnp.float32),
            pltpu.VMEM((N,), jnp.float32),
            pltpu.VMEM((N,), jnp.float32),
            pltpu.SemaphoreType.DMA,
            pltpu.SemaphoreType.DMA,
            pltpu.SemaphoreType.DMA,
            pltpu.SemaphoreType.DMA,
        ],
    )


def kernel(k, v, scores, n_evict):
    del n_evict
    _, sidx = _sort_scores(scores)
    k6 = jnp.transpose(k.reshape(B, H, NT, 128, DT, 8), (0, 1, 4, 2, 5, 3))
    v6 = jnp.transpose(v.reshape(B, H, NT, 128, DT, 8), (0, 1, 4, 2, 5, 3))
    kk6, kv6, ek6, ev6, ks, es = _make_sc_gather()(k6, v6, sidx, scores)

    def back(x6, S):
        return jnp.transpose(x6, (0, 1, 3, 5, 2, 4)).reshape(B, H, S, D)

    return (back(kk6, KEEP), back(kv6, KEEP), ks,
            back(ek6, EVICT), back(ev6, EVICT), es)


# speedup vs baseline: 5.2428x; 1.0047x over previous
import jax
import jax.numpy as jnp
from jax import lax
from jax.experimental import pallas as pl
from jax.experimental.pallas import tpu as pltpu
from jax.experimental.pallas import tpu_sc as plsc

B, H, N, D = 8, 16, 8192, 64
KEEP = 6144
EVICT = 2048
NT = N // 128
KT = KEEP // 128
ET = EVICT // 128
DT = D // 8


def _sort_body(jt_ref, kt_ref, s_in_ref, ss_ref, si_ref):
    step = pl.program_id(0)

    @pl.when(step == 0)
    def _():
        ss_ref[...] = s_in_ref[...]
        si_ref[...] = lax.broadcasted_iota(jnp.int32, (B, N), 1)

    j = jt_ref[step]
    k = kt_ref[step]
    s = ss_ref[...]
    ix = si_ref[...]
    iota = lax.broadcasted_iota(jnp.int32, (B, N), 1)
    bitj = (iota & j) == 0
    sp = jnp.where(bitj, pltpu.roll(s, N - j, 1), pltpu.roll(s, j, 1))
    ip = jnp.where(bitj, pltpu.roll(ix, N - j, 1), pltpu.roll(ix, j, 1))
    up = (iota & k) == 0
    lt_peer = (sp > s) | ((sp == s) & (ip < ix))
    take = lt_peer == (bitj == up)
    ss_ref[...] = jnp.where(take, sp, s)
    si_ref[...] = jnp.where(take, ip, ix)


def _bitonic_steps():
    js, ks = [], []
    k = 2
    while k <= N:
        j = k // 2
        while j >= 1:
            js.append(j)
            ks.append(k)
            j //= 2
        k *= 2
    return js, ks


def _sort_scores(scores):
    js, ks = _bitonic_steps()
    jt = jnp.asarray(js, dtype=jnp.int32)
    kt = jnp.asarray(ks, dtype=jnp.int32)
    ss, si = pl.pallas_call(
        _sort_body,
        grid=(len(js),),
        in_specs=[
            pl.BlockSpec(memory_space=pltpu.SMEM),
            pl.BlockSpec(memory_space=pltpu.SMEM),
            pl.BlockSpec((B, N), lambda i: (0, 0)),
        ],
        out_specs=[
            pl.BlockSpec((B, N), lambda i: (0, 0)),
            pl.BlockSpec((B, N), lambda i: (0, 0)),
        ],
        out_shape=[
            jax.ShapeDtypeStruct((B, N), jnp.float32),
            jax.ShapeDtypeStruct((B, N), jnp.int32),
        ],
    )(jt, kt, scores)
    return ss, si


NC, NS = 2, 16
NW = NC * NS
TPW = (B * H) // NW
LANES = 16
TOTR = TPW * D
UNROLL = 16


def _sc_body(k6, v6, sidx_hbm, sc_hbm,
             kk6, kv6, ek6, ev6, ks, es,
             sidx_v, krow0, krow1, vrow0, vrow1,
             kob0, kob1, vob0, vob1, srow_v, sob_v,
             rsem0, rsem1, wsem0, wsem1):
    c = lax.axis_index("c")
    s = lax.axis_index("s")
    wid = s * NC + c
    b = wid // TPW

    krow = (krow0, krow1)
    vrow = (vrow0, vrow1)
    kob = (kob0, kob1)
    vob = (vob0, vob1)
    rsem = (rsem0, rsem1)
    wsem = (wsem0, wsem1)

    pltpu.sync_copy(sidx_hbm.at[b], sidx_v)

    def rowrefs(r):
        t = r // D
        d = r % D
        bh = wid * TPW + t
        return bh // H, bh % H, d // 8, d % 8

    def prefetch(r, p):
        b4, h4, dt, dr = rowrefs(r)
        pltpu.async_copy(k6.at[b4, h4, dt, :, dr], krow[p], rsem[p])
        pltpu.async_copy(v6.at[b4, h4, dt, :, dr], vrow[p], rsem[p])

    prefetch(0, 0)

    def half(r, p):
        b4, h4, dt, dr = rowrefs(r)

        @pl.when(r < TOTR - 1)
        def _():
            prefetch(r + 1, 1 - p)

        pltpu.make_async_copy(k6.at[0, 0, 0, :, 0], krow[p], rsem[p]).wait()
        pltpu.make_async_copy(k6.at[0, 0, 0, :, 0], vrow[p], rsem[p]).wait()

        @pl.when(r >= 2)
        def _():
            pltpu.make_async_copy(kob[p], kk6.at[0, 0, 0, :, 0], wsem[p]).wait()
            pltpu.make_async_copy(kob[p], ek6.at[0, 0, 0, :, 0], wsem[p]).wait()
            pltpu.make_async_copy(vob[p], kk6.at[0, 0, 0, :, 0], wsem[p]).wait()
            pltpu.make_async_copy(vob[p], ek6.at[0, 0, 0, :, 0], wsem[p]).wait()

        @plsc.parallel_loop(0, N, step=LANES, unroll=UNROLL)
        def _(o):
            sl = pl.ds(o, LANES)
            iv = sidx_v[sl]
            hi = iv >> 7
            lo = iv & 127
            r2 = o // 128
            c2 = o % 128
            cs = pl.ds(c2, LANES)
            kob[p][r2, cs] = plsc.load_gather(krow[p], [hi, lo])
            vob[p][r2, cs] = plsc.load_gather(vrow[p], [hi, lo])

        pltpu.async_copy(kob[p].at[pl.ds(0, KT)], kk6.at[b4, h4, dt, :, dr],
                         wsem[p])
        pltpu.async_copy(kob[p].at[pl.ds(KT, ET)], ek6.at[b4, h4, dt, :, dr],
                         wsem[p])
        pltpu.async_copy(vob[p].at[pl.ds(0, KT)], kv6.at[b4, h4, dt, :, dr],
                         wsem[p])
        pltpu.async_copy(vob[p].at[pl.ds(KT, ET)], ev6.at[b4, h4, dt, :, dr],
                         wsem[p])

    def outer(r2, carry):
        half(r2 * 2, 0)
        half(r2 * 2 + 1, 1)
        return carry

    lax.fori_loop(0, TOTR // 2, outer, 0)

    @pl.when(wid % TPW == 0)
    def _():
        pltpu.sync_copy(sc_hbm.at[b], srow_v)

        @plsc.parallel_loop(0, N, step=LANES, unroll=UNROLL)
        def _(o):
            sl = pl.ds(o, LANES)
            sob_v[sl] = plsc.load_gather(srow_v, [sidx_v[sl]])

        pltpu.sync_copy(sob_v.at[pl.ds(0, KEEP)], ks.at[b])
        pltpu.sync_copy(sob_v.at[pl.ds(KEEP, EVICT)], es.at[b])

    for p in range(2):
        pltpu.make_async_copy(kob[p], kk6.at[0, 0, 0, :, 0], wsem[p]).wait()
        pltpu.make_async_copy(kob[p], ek6.at[0, 0, 0, :, 0], wsem[p]).wait()
        pltpu.make_async_copy(vob[p], kk6.at[0, 0, 0, :, 0], wsem[p]).wait()
        pltpu.make_async_copy(vob[p], ek6.at[0, 0, 0, :, 0], wsem[p]).wait()


def _make_sc_gather():
    return pl.kernel(
        _sc_body,
        out_type=(
            jax.ShapeDtypeStruct((B, H, DT, KT, 8, 128), jnp.float32),
            jax.ShapeDtypeStruct((B, H, DT, KT, 8, 128), jnp.float32),
            jax.ShapeDtypeStruct((B, H, DT, ET, 8, 128), jnp.float32),
            jax.ShapeDtypeStruct((B, H, DT, ET, 8, 128), jnp.float32),
            jax.ShapeDtypeStruct((B, KEEP), jnp.float32),
            jax.ShapeDtypeStruct((B, EVICT), jnp.float32),
        ),
        mesh=plsc.VectorSubcoreMesh(
            core_axis_name="c", subcore_axis_name="s",
            num_cores=NC, num_subcores=NS),
        compiler_params=pltpu.CompilerParams(
            use_tc_tiling_on_sc=False, needs_layout_passes=False),
        scratch_types=[
            pltpu.VMEM((N,), jnp.int32),
            pltpu.VMEM((NT, 128), jnp.float32),
            pltpu.VMEM((NT, 128), jnp.float32),
            pltpu.VMEM((NT, 128), jnp.float32),
            pltpu.VMEM((NT, 128), jnp.float32),
            pltpu.VMEM((NT, 128), jnp.float32),
            pltpu.VMEM((NT, 128), jnp.float32),
            pltpu.VMEM((NT, 128), jnp.float32),
            pltpu.VMEM((NT, 128), jnp.float32),
            pltpu.VMEM((N,), jnp.float32),
            pltpu.VMEM((N,), jnp.float32),
            pltpu.SemaphoreType.DMA,
            pltpu.SemaphoreType.DMA,
            pltpu.SemaphoreType.DMA,
            pltpu.SemaphoreType.DMA,
        ],
    )


def kernel(k, v, scores, n_evict):
    del n_evict
    _, sidx = _sort_scores(scores)
    k6 = jnp.transpose(k.reshape(B, H, NT, 128, DT, 8), (0, 1, 4, 2, 5, 3))
    v6 = jnp.transpose(v.reshape(B, H, NT, 128, DT, 8), (0, 1, 4, 2, 5, 3))
    kk6, kv6, ek6, ev6, ks, es = _make_sc_gather()(k6, v6, sidx, scores)

    def back(x6, S):
        return jnp.transpose(x6, (0, 1, 3, 5, 2, 4)).reshape(B, H, S, D)

    return (back(kk6, KEEP), back(kv6, KEEP), ks,
            back(ek6, EVICT), back(ev6, EVICT), es)


# speedup vs baseline: 5.4747x; 1.0442x over previous
import jax
import jax.numpy as jnp
from jax import lax
from jax.experimental import pallas as pl
from jax.experimental.pallas import tpu as pltpu
from jax.experimental.pallas import tpu_sc as plsc

B, H, N, D = 8, 16, 8192, 64
KEEP = 6144
EVICT = 2048
NT = N // 128
KT = KEEP // 128
ET = EVICT // 128
DT = D // 8


FUSE = 7


def _sort_body(jt_ref, kt_ref, s_in_ref, ss_ref, si_ref):
    step = pl.program_id(0)

    @pl.when(step == 0)
    def _():
        ss_ref[...] = s_in_ref[...]
        si_ref[...] = lax.broadcasted_iota(jnp.int32, (B, N), 1)

    s = ss_ref[...]
    ix = si_ref[...]
    iota = lax.broadcasted_iota(jnp.int32, (B, N), 1)
    for u in range(FUSE):
        j = jt_ref[step * FUSE + u]
        k = kt_ref[step * FUSE + u]
        bitj = (iota & j) == 0
        sp = jnp.where(bitj, pltpu.roll(s, N - j, 1), pltpu.roll(s, j, 1))
        ip = jnp.where(bitj, pltpu.roll(ix, N - j, 1), pltpu.roll(ix, j, 1))
        up = (iota & k) == 0
        lt_peer = (sp > s) | ((sp == s) & (ip < ix))
        take = lt_peer == (bitj == up)
        s = jnp.where(take, sp, s)
        ix = jnp.where(take, ip, ix)
    ss_ref[...] = s
    si_ref[...] = ix


def _bitonic_steps():
    js, ks = [], []
    k = 2
    while k <= N:
        j = k // 2
        while j >= 1:
            js.append(j)
            ks.append(k)
            j //= 2
        k *= 2
    return js, ks


def _sort_scores(scores):
    js, ks = _bitonic_steps()
    jt = jnp.asarray(js, dtype=jnp.int32)
    kt = jnp.asarray(ks, dtype=jnp.int32)
    assert len(js) % FUSE == 0
    ss, si = pl.pallas_call(
        _sort_body,
        grid=(len(js) // FUSE,),
        in_specs=[
            pl.BlockSpec(memory_space=pltpu.SMEM),
            pl.BlockSpec(memory_space=pltpu.SMEM),
            pl.BlockSpec((B, N), lambda i: (0, 0)),
        ],
        out_specs=[
            pl.BlockSpec((B, N), lambda i: (0, 0)),
            pl.BlockSpec((B, N), lambda i: (0, 0)),
        ],
        out_shape=[
            jax.ShapeDtypeStruct((B, N), jnp.float32),
            jax.ShapeDtypeStruct((B, N), jnp.int32),
        ],
    )(jt, kt, scores)
    return ss, si


NC, NS = 2, 16
NW = NC * NS
TPW = (B * H) // NW
LANES = 16
TOTR = TPW * D
UNROLL = 16


def _sc_body(k6, v6, sidx_hbm, sc_hbm,
             kk6, kv6, ek6, ev6, ks, es,
             sidx_v, krow0, krow1, vrow0, vrow1,
             kob0, kob1, vob0, vob1, srow_v, sob_v,
             rsem0, rsem1, wsem0, wsem1):
    c = lax.axis_index("c")
    s = lax.axis_index("s")
    wid = s * NC + c
    b = wid // TPW

    krow = (krow0, krow1)
    vrow = (vrow0, vrow1)
    kob = (kob0, kob1)
    vob = (vob0, vob1)
    rsem = (rsem0, rsem1)
    wsem = (wsem0, wsem1)

    pltpu.sync_copy(sidx_hbm.at[b], sidx_v)

    def rowrefs(r):
        t = r // D
        d = r % D
        bh = wid * TPW + t
        return bh // H, bh % H, d // 8, d % 8

    def prefetch(r, p):
        b4, h4, dt, dr = rowrefs(r)
        pltpu.async_copy(k6.at[b4, h4, dt, :, dr], krow[p], rsem[p])
        pltpu.async_copy(v6.at[b4, h4, dt, :, dr], vrow[p], rsem[p])

    prefetch(0, 0)

    def half(r, p):
        b4, h4, dt, dr = rowrefs(r)

        @pl.when(r < TOTR - 1)
        def _():
            prefetch(r + 1, 1 - p)

        pltpu.make_async_copy(k6.at[0, 0, 0, :, 0], krow[p], rsem[p]).wait()
        pltpu.make_async_copy(k6.at[0, 0, 0, :, 0], vrow[p], rsem[p]).wait()

        @pl.when(r >= 2)
        def _():
            pltpu.make_async_copy(kob[p], kk6.at[0, 0, 0, :, 0], wsem[p]).wait()
            pltpu.make_async_copy(kob[p], ek6.at[0, 0, 0, :, 0], wsem[p]).wait()
            pltpu.make_async_copy(vob[p], kk6.at[0, 0, 0, :, 0], wsem[p]).wait()
            pltpu.make_async_copy(vob[p], ek6.at[0, 0, 0, :, 0], wsem[p]).wait()

        @plsc.parallel_loop(0, N, step=LANES, unroll=UNROLL)
        def _(o):
            sl = pl.ds(o, LANES)
            iv = sidx_v[sl]
            hi = iv >> 7
            lo = iv & 127
            r2 = o // 128
            c2 = o % 128
            cs = pl.ds(c2, LANES)
            kob[p][r2, cs] = plsc.load_gather(krow[p], [hi, lo])
            vob[p][r2, cs] = plsc.load_gather(vrow[p], [hi, lo])

        pltpu.async_copy(kob[p].at[pl.ds(0, KT)], kk6.at[b4, h4, dt, :, dr],
                         wsem[p])
        pltpu.async_copy(kob[p].at[pl.ds(KT, ET)], ek6.at[b4, h4, dt, :, dr],
                         wsem[p])
        pltpu.async_copy(vob[p].at[pl.ds(0, KT)], kv6.at[b4, h4, dt, :, dr],
                         wsem[p])
        pltpu.async_copy(vob[p].at[pl.ds(KT, ET)], ev6.at[b4, h4, dt, :, dr],
                         wsem[p])

    def outer(r2, carry):
        half(r2 * 2, 0)
        half(r2 * 2 + 1, 1)
        return carry

    lax.fori_loop(0, TOTR // 2, outer, 0)

    @pl.when(wid % TPW == 0)
    def _():
        pltpu.sync_copy(sc_hbm.at[b], srow_v)

        @plsc.parallel_loop(0, N, step=LANES, unroll=UNROLL)
        def _(o):
            sl = pl.ds(o, LANES)
            sob_v[sl] = plsc.load_gather(srow_v, [sidx_v[sl]])

        pltpu.sync_copy(sob_v.at[pl.ds(0, KEEP)], ks.at[b])
        pltpu.sync_copy(sob_v.at[pl.ds(KEEP, EVICT)], es.at[b])

    for p in range(2):
        pltpu.make_async_copy(kob[p], kk6.at[0, 0, 0, :, 0], wsem[p]).wait()
        pltpu.make_async_copy(kob[p], ek6.at[0, 0, 0, :, 0], wsem[p]).wait()
        pltpu.make_async_copy(vob[p], kk6.at[0, 0, 0, :, 0], wsem[p]).wait()
        pltpu.make_async_copy(vob[p], ek6.at[0, 0, 0, :, 0], wsem[p]).wait()


def _make_sc_gather():
    return pl.kernel(
        _sc_body,
        out_type=(
            jax.ShapeDtypeStruct((B, H, DT, KT, 8, 128), jnp.float32),
            jax.ShapeDtypeStruct((B, H, DT, KT, 8, 128), jnp.float32),
            jax.ShapeDtypeStruct((B, H, DT, ET, 8, 128), jnp.float32),
            jax.ShapeDtypeStruct((B, H, DT, ET, 8, 128), jnp.float32),
            jax.ShapeDtypeStruct((B, KEEP), jnp.float32),
            jax.ShapeDtypeStruct((B, EVICT), jnp.float32),
        ),
        mesh=plsc.VectorSubcoreMesh(
            core_axis_name="c", subcore_axis_name="s",
            num_cores=NC, num_subcores=NS),
        compiler_params=pltpu.CompilerParams(
            use_tc_tiling_on_sc=False, needs_layout_passes=False),
        scratch_types=[
            pltpu.VMEM((N,), jnp.int32),
            pltpu.VMEM((NT, 128), jnp.float32),
            pltpu.VMEM((NT, 128), jnp.float32),
            pltpu.VMEM((NT, 128), jnp.float32),
            pltpu.VMEM((NT, 128), jnp.float32),
            pltpu.VMEM((NT, 128), jnp.float32),
            pltpu.VMEM((NT, 128), jnp.float32),
            pltpu.VMEM((NT, 128), jnp.float32),
            pltpu.VMEM((NT, 128), jnp.float32),
            pltpu.VMEM((N,), jnp.float32),
            pltpu.VMEM((N,), jnp.float32),
            pltpu.SemaphoreType.DMA,
            pltpu.SemaphoreType.DMA,
            pltpu.SemaphoreType.DMA,
            pltpu.SemaphoreType.DMA,
        ],
    )


def kernel(k, v, scores, n_evict):
    del n_evict
    _, sidx = _sort_scores(scores)
    k6 = jnp.transpose(k.reshape(B, H, NT, 128, DT, 8), (0, 1, 4, 2, 5, 3))
    v6 = jnp.transpose(v.reshape(B, H, NT, 128, DT, 8), (0, 1, 4, 2, 5, 3))
    kk6, kv6, ek6, ev6, ks, es = _make_sc_gather()(k6, v6, sidx, scores)

    def back(x6, S):
        return jnp.transpose(x6, (0, 1, 3, 5, 2, 4)).reshape(B, H, S, D)

    return (back(kk6, KEEP), back(kv6, KEEP), ks,
            back(ek6, EVICT), back(ev6, EVICT), es)


# speedup vs baseline: 5.7459x; 1.0496x over previous
import jax
import jax.numpy as jnp
from jax import lax
from jax.experimental import pallas as pl
from jax.experimental.pallas import tpu as pltpu
from jax.experimental.pallas import tpu_sc as plsc

B, H, N, D = 8, 16, 8192, 64
KEEP = 6144
EVICT = 2048
NT = N // 128
KT = KEEP // 128
ET = EVICT // 128
DT = D // 8


FUSE = 7


def _sort_body(jt_ref, kt_ref, s_in_ref, ss_ref, si_ref):
    step = pl.program_id(0)

    @pl.when(step == 0)
    def _():
        ss_ref[...] = s_in_ref[...]
        si_ref[...] = lax.broadcasted_iota(jnp.int32, (B, N), 1)

    s = ss_ref[...]
    ix = si_ref[...]
    iota = lax.broadcasted_iota(jnp.int32, (B, N), 1)
    for u in range(FUSE):
        j = jt_ref[step * FUSE + u]
        k = kt_ref[step * FUSE + u]
        bitj = (iota & j) == 0
        sp = jnp.where(bitj, pltpu.roll(s, N - j, 1), pltpu.roll(s, j, 1))
        ip = jnp.where(bitj, pltpu.roll(ix, N - j, 1), pltpu.roll(ix, j, 1))
        up = (iota & k) == 0
        lt_peer = (sp > s) | ((sp == s) & (ip < ix))
        take = lt_peer == (bitj == up)
        s = jnp.where(take, sp, s)
        ix = jnp.where(take, ip, ix)
    ss_ref[...] = s
    si_ref[...] = ix


def _bitonic_steps():
    js, ks = [], []
    k = 2
    while k <= N:
        j = k // 2
        while j >= 1:
            js.append(j)
            ks.append(k)
            j //= 2
        k *= 2
    return js, ks


def _sort_scores(scores):
    js, ks = _bitonic_steps()
    jt = jnp.asarray(js, dtype=jnp.int32)
    kt = jnp.asarray(ks, dtype=jnp.int32)
    assert len(js) % FUSE == 0
    ss, si = pl.pallas_call(
        _sort_body,
        grid=(len(js) // FUSE,),
        in_specs=[
            pl.BlockSpec(memory_space=pltpu.SMEM),
            pl.BlockSpec(memory_space=pltpu.SMEM),
            pl.BlockSpec((B, N), lambda i: (0, 0)),
        ],
        out_specs=[
            pl.BlockSpec((B, N), lambda i: (0, 0)),
            pl.BlockSpec((B, N), lambda i: (0, 0)),
        ],
        out_shape=[
            jax.ShapeDtypeStruct((B, N), jnp.float32),
            jax.ShapeDtypeStruct((B, N), jnp.int32),
        ],
    )(jt, kt, scores)
    return ss, si


NC, NS = 2, 16
NW = NC * NS
TPW = (B * H) // NW
LANES = 16
TOTR = TPW * D
UNROLL = 16


def _sc_body(k6, v6, sidx_hbm, sc_hbm,
             kk6, kv6, ek6, ev6, ks, es,
             sidx_v, krow0, krow1, vrow0, vrow1,
             kob0, kob1, vob0, vob1, srow_v, sob_v,
             rsem0, rsem1, wsem0, wsem1):
    c = lax.axis_index("c")
    s = lax.axis_index("s")
    wid = s * NC + c
    b = wid // TPW

    krow = (krow0, krow1)
    vrow = (vrow0, vrow1)
    kob = (kob0, kob1)
    vob = (vob0, vob1)
    rsem = (rsem0, rsem1)
    wsem = (wsem0, wsem1)

    pltpu.sync_copy(sidx_hbm.at[b], sidx_v)

    def pairrefs(q):
        t = q // (D // 2)
        dq = q % (D // 2)
        bh = wid * TPW + t
        return bh // H, bh % H, dq // 4, dq % 4

    def prefetch(q, pp):
        b4, h4, dt, dp = pairrefs(q)
        pltpu.async_copy(k6.at[b4, h4, dt, :, pl.ds(dp * 2, 2)],
                         krow[pp], rsem[pp])
        pltpu.async_copy(v6.at[b4, h4, dt, :, pl.ds(dp * 2, 2)],
                         vrow[pp], rsem[pp])

    prefetch(0, 0)

    def half(q, pp):
        b4, h4, dt, dp = pairrefs(q)

        @pl.when(q < TOTR // 2 - 1)
        def _():
            prefetch(q + 1, 1 - pp)

        pltpu.make_async_copy(k6.at[0, 0, 0, :, pl.ds(0, 2)], krow[pp],
                              rsem[pp]).wait()
        pltpu.make_async_copy(k6.at[0, 0, 0, :, pl.ds(0, 2)], vrow[pp],
                              rsem[pp]).wait()

        for sub in range(2):
            dr = dp * 2 + sub

            @pl.when(q >= 1)
            def _(sub=sub):
                pltpu.make_async_copy(kob[sub], kk6.at[0, 0, 0, :, 0],
                                      wsem[sub]).wait()
                pltpu.make_async_copy(kob[sub], ek6.at[0, 0, 0, :, 0],
                                      wsem[sub]).wait()
                pltpu.make_async_copy(vob[sub], kk6.at[0, 0, 0, :, 0],
                                      wsem[sub]).wait()
                pltpu.make_async_copy(vob[sub], ek6.at[0, 0, 0, :, 0],
                                      wsem[sub]).wait()

            @plsc.parallel_loop(0, N, step=LANES, unroll=UNROLL)
            def _(o, sub=sub):
                sl = pl.ds(o, LANES)
                iv = sidx_v[sl]
                hi = iv >> 7
                lo = iv & 127
                sv = jnp.full((LANES,), sub, jnp.int32)
                r2 = o // 128
                c2 = o % 128
                cs = pl.ds(c2, LANES)
                kob[sub][r2, cs] = plsc.load_gather(krow[pp], [hi, sv, lo])
                vob[sub][r2, cs] = plsc.load_gather(vrow[pp], [hi, sv, lo])

            pltpu.async_copy(kob[sub].at[pl.ds(0, KT)],
                             kk6.at[b4, h4, dt, :, dr], wsem[sub])
            pltpu.async_copy(kob[sub].at[pl.ds(KT, ET)],
                             ek6.at[b4, h4, dt, :, dr], wsem[sub])
            pltpu.async_copy(vob[sub].at[pl.ds(0, KT)],
                             kv6.at[b4, h4, dt, :, dr], wsem[sub])
            pltpu.async_copy(vob[sub].at[pl.ds(KT, ET)],
                             ev6.at[b4, h4, dt, :, dr], wsem[sub])

    def outer(q2, carry):
        half(q2 * 2, 0)
        half(q2 * 2 + 1, 1)
        return carry

    lax.fori_loop(0, TOTR // 4, outer, 0)

    @pl.when(wid % TPW == 0)
    def _():
        pltpu.sync_copy(sc_hbm.at[b], srow_v)

        @plsc.parallel_loop(0, N, step=LANES, unroll=UNROLL)
        def _(o):
            sl = pl.ds(o, LANES)
            sob_v[sl] = plsc.load_gather(srow_v, [sidx_v[sl]])

        pltpu.sync_copy(sob_v.at[pl.ds(0, KEEP)], ks.at[b])
        pltpu.sync_copy(sob_v.at[pl.ds(KEEP, EVICT)], es.at[b])

    for p in range(2):
        pltpu.make_async_copy(kob[p], kk6.at[0, 0, 0, :, 0], wsem[p]).wait()
        pltpu.make_async_copy(kob[p], ek6.at[0, 0, 0, :, 0], wsem[p]).wait()
        pltpu.make_async_copy(vob[p], kk6.at[0, 0, 0, :, 0], wsem[p]).wait()
        pltpu.make_async_copy(vob[p], ek6.at[0, 0, 0, :, 0], wsem[p]).wait()


def _make_sc_gather():
    return pl.kernel(
        _sc_body,
        out_type=(
            jax.ShapeDtypeStruct((B, H, DT, KT, 8, 128), jnp.float32),
            jax.ShapeDtypeStruct((B, H, DT, KT, 8, 128), jnp.float32),
            jax.ShapeDtypeStruct((B, H, DT, ET, 8, 128), jnp.float32),
            jax.ShapeDtypeStruct((B, H, DT, ET, 8, 128), jnp.float32),
            jax.ShapeDtypeStruct((B, KEEP), jnp.float32),
            jax.ShapeDtypeStruct((B, EVICT), jnp.float32),
        ),
        mesh=plsc.VectorSubcoreMesh(
            core_axis_name="c", subcore_axis_name="s",
            num_cores=NC, num_subcores=NS),
        compiler_params=pltpu.CompilerParams(
            use_tc_tiling_on_sc=False, needs_layout_passes=False),
        scratch_types=[
            pltpu.VMEM((N,), jnp.int32),
            pltpu.VMEM((NT, 2, 128), jnp.float32),
            pltpu.VMEM((NT, 2, 128), jnp.float32),
            pltpu.VMEM((NT, 2, 128), jnp.float32),
            pltpu.VMEM((NT, 2, 128), jnp.float32),
            pltpu.VMEM((NT, 128), jnp.float32),
            pltpu.VMEM((NT, 128), jnp.float32),
            pltpu.VMEM((NT, 128), jnp.float32),
            pltpu.VMEM((NT, 128), jnp.float32),
            pltpu.VMEM((N,), jnp.float32),
            pltpu.VMEM((N,), jnp.float32),
            pltpu.SemaphoreType.DMA,
            pltpu.SemaphoreType.DMA,
            pltpu.SemaphoreType.DMA,
            pltpu.SemaphoreType.DMA,
        ],
    )


def kernel(k, v, scores, n_evict):
    del n_evict
    _, sidx = _sort_scores(scores)
    k6 = jnp.transpose(k.reshape(B, H, NT, 128, DT, 8), (0, 1, 4, 2, 5, 3))
    v6 = jnp.transpose(v.reshape(B, H, NT, 128, DT, 8), (0, 1, 4, 2, 5, 3))
    kk6, kv6, ek6, ev6, ks, es = _make_sc_gather()(k6, v6, sidx, scores)

    def back(x6, S):
        return jnp.transpose(x6, (0, 1, 3, 5, 2, 4)).reshape(B, H, S, D)

    return (back(kk6, KEEP), back(kv6, KEEP), ks,
            back(ek6, EVICT), back(ev6, EVICT), es)
